# baseline (speedup 1.0000x reference)
import functools

import jax
import jax.numpy as jnp
from jax import lax
from jax.experimental import pallas as pl
from jax.experimental.pallas import tpu as pltpu
from jax.experimental.pallas import tpu_sc as plsc

_D = 1000
_BATCH = 4096
_T = 20
_NBC = 2


def _build():
    info = plsc.get_sparse_core_info()
    nc = info.num_cores
    nw = nc * info.num_subcores
    nb_w = _BATCH // nw
    n_chunks = nb_w // _NBC
    mesh = plsc.VectorSubcoreMesh(core_axis_name="c", subcore_axis_name="s")

    @functools.partial(
        pl.kernel,
        mesh=mesh,
        out_type=jax.ShapeDtypeStruct((_BATCH, _T, _D), jnp.float32),
        scratch_types=[
            pltpu.VMEM((nb_w, _T), jnp.int32),
            pltpu.VMEM((2, _NBC, _T, _D), jnp.float32),
            pltpu.SemaphoreType.DMA,
            pltpu.SemaphoreType.DMA,
        ],
        compiler_params=pltpu.CompilerParams(use_tc_tiling_on_sc=False),
    )
    def emb(x_hbm, table_hbm, out_hbm, idx_v, rows_v, gsem, osem):
        wid = lax.axis_index("s") * nc + lax.axis_index("c")
        base = wid * nb_w
        pltpu.sync_copy(x_hbm.at[pl.ds(base, nb_w)], idx_v)

        def gather(c, slot):
            for i in range(_NBC):
                pltpu.async_copy(
                    table_hbm.at[idx_v.at[c * _NBC + i]],
                    rows_v.at[slot, i], gsem)

        def wait_gather(slot):
            for i in range(_NBC):
                pltpu.make_async_copy(
                    table_hbm.at[idx_v.at[0]],
                    rows_v.at[slot, i], gsem).wait()

        def put(c, slot):
            pltpu.async_copy(
                rows_v.at[slot],
                out_hbm.at[pl.ds(base + c * _NBC, _NBC)], osem)

        def wait_put(slot):
            pltpu.make_async_copy(
                rows_v.at[slot],
                out_hbm.at[pl.ds(base, _NBC)], osem).wait()

        gather(0, 0)

        def pair(g, _):
            for b in range(2):
                c = 2 * g + b
                wait_gather(b)

                @pl.when(c >= 1)
                def _():
                    wait_put(1 - b)

                @pl.when(c + 1 < n_chunks)
                def _():
                    gather(c + 1, 1 - b)

                put(c, b)
            return 0

        lax.fori_loop(0, n_chunks // 2, pair, 0)
        wait_put(1)

    return emb


_emb = _build()


def kernel(x, table):
    return _emb(x.astype(jnp.int32), table)


# speedup vs baseline: 1.0187x; 1.0187x over previous
import functools

import jax
import jax.numpy as jnp
from jax import lax
from jax.experimental import pallas as pl
from jax.experimental.pallas import tpu as pltpu
from jax.experimental.pallas import tpu_sc as plsc

_T = 20
_NDT = 125
_NBT = 32
_BATCH = 4096


def _build():
    info = plsc.get_sparse_core_info()
    nc = info.num_cores
    nw = nc * info.num_subcores
    mesh = plsc.VectorSubcoreMesh(core_axis_name="c", subcore_axis_name="s")

    @functools.partial(
        pl.kernel,
        mesh=mesh,
        out_type=jax.ShapeDtypeStruct((_T, _NDT, _NBT, 8, 128), jnp.float32),
        scratch_types=[
            pltpu.VMEM((_T, _BATCH), jnp.int32),
            pltpu.VMEM((8, 1000), jnp.float32),
            pltpu.VMEM((2, 16, 8, 128), jnp.float32),
            pltpu.SemaphoreType.DMA,
            pltpu.SemaphoreType.DMA,
        ],
        compiler_params=pltpu.CompilerParams(
            use_tc_tiling_on_sc=False, needs_layout_passes=False),
    )
    def emb(xt_hbm, tabr_hbm, out_hbm, x_v, slice_v, blk_v, ssem, osem):
        wid = lax.axis_index("s") * nc + lax.axis_index("c")
        lo = wid * _NDT // nw
        hi = (wid + 1) * _NDT // nw
        pltpu.sync_copy(xt_hbm, x_v)

        ds_idx = [jnp.full((16,), ds, jnp.int32) for ds in range(8)]

        def wait_put(slot):
            pltpu.make_async_copy(
                blk_v.at[slot],
                out_hbm.at[0, 0, pl.ds(0, 16)], osem).wait()

        def dt_body(dt, _):
            i = dt - lo
            pltpu.async_copy(tabr_hbm.at[dt], slice_v, ssem).wait()

            def t_body(t, _):
                for h in range(2):
                    g = (i * _T + t) * 2 + h

                    @pl.when(g >= 2)
                    def _():
                        wait_put(h)

                    def bt_body(btl, _):
                        for bl in range(8):
                            xv = x_v[t, pl.ds(h * 2048 + btl * 128 + bl * 16, 16)]
                            for ds in range(8):
                                vals = plsc.load_gather(slice_v, [ds_idx[ds], xv])
                                blk_v[h, btl, ds, pl.ds(bl * 16, 16)] = vals
                        return 0

                    lax.fori_loop(0, 16, bt_body, 0)
                    pltpu.async_copy(
                        blk_v.at[h],
                        out_hbm.at[t, dt, pl.ds(h * 16, 16)], osem)
                return 0

            lax.fori_loop(0, _T, t_body, 0)
            return 0

        lax.fori_loop(lo, hi, dt_body, 0)
        wait_put(0)
        wait_put(1)

    return emb


_emb = _build()


def kernel(x, table):
    xt = x.T.astype(jnp.int32)
    tabr = table.T.reshape(_NDT, 8, 1000)
    out5 = _emb(xt, tabr)
    return jnp.transpose(out5, (2, 4, 0, 1, 3)).reshape(_BATCH, _T, _NDT * 8)


# speedup vs baseline: 2.6953x; 2.6457x over previous
import functools

import jax
import jax.numpy as jnp
from jax import lax
from jax.experimental import pallas as pl
from jax.experimental.pallas import tpu as pltpu
from jax.experimental.pallas import tpu_sc as plsc

_T = 20
_NDT = 125
_NBT = 32
_BATCH = 4096


def _build():
    info = plsc.get_sparse_core_info()
    nc = info.num_cores
    nw = nc * info.num_subcores
    mesh = plsc.VectorSubcoreMesh(core_axis_name="c", subcore_axis_name="s")

    @functools.partial(
        pl.kernel,
        mesh=mesh,
        out_type=jax.ShapeDtypeStruct((_T, _NDT, _NBT, 8, 128), jnp.float32),
        scratch_types=[
            pltpu.VMEM((_T, _BATCH), jnp.int32),
            pltpu.VMEM((8000,), jnp.float32),
            pltpu.VMEM((2, 16, 8, 128), jnp.float32),
            pltpu.SemaphoreType.DMA,
            pltpu.SemaphoreType.DMA,
        ],
        compiler_params=pltpu.CompilerParams(
            use_tc_tiling_on_sc=False, needs_layout_passes=False),
    )
    def emb(xt_hbm, tabr_hbm, out_hbm, x_v, slice_v, blk_v, ssem, osem):
        wid = lax.axis_index("s") * nc + lax.axis_index("c")
        lo = wid * _NDT // nw
        hi = (wid + 1) * _NDT // nw
        pltpu.sync_copy(xt_hbm, x_v)

        ds_off = [jnp.full((16,), ds * 1000, jnp.int32) for ds in range(8)]

        def wait_put(slot):
            pltpu.make_async_copy(
                blk_v.at[slot],
                out_hbm.at[0, 0, pl.ds(0, 16)], osem).wait()

        def dt_body(dt, _):
            i = dt - lo
            pltpu.async_copy(tabr_hbm.at[dt], slice_v, ssem).wait()

            def t_body(t, _):
                for h in range(2):
                    g = (i * _T + t) * 2 + h

                    @pl.when(g >= 2)
                    def _():
                        wait_put(h)

                    @plsc.parallel_loop(0, 16, unroll=2)
                    def bt_body(btl):
                        for bl in range(8):
                            xv = x_v[t, pl.ds(h * 2048 + btl * 128 + bl * 16, 16)]
                            for ds in range(8):
                                vals = plsc.load_gather(
                                    slice_v, [xv + ds_off[ds]])
                                blk_v[h, btl, ds, pl.ds(bl * 16, 16)] = vals
                    pltpu.async_copy(
                        blk_v.at[h],
                        out_hbm.at[t, dt, pl.ds(h * 16, 16)], osem)
                return 0

            lax.fori_loop(0, _T, t_body, 0)
            return 0

        lax.fori_loop(lo, hi, dt_body, 0)
        wait_put(0)
        wait_put(1)

    return emb


_emb = _build()


def kernel(x, table):
    xt = x.T.astype(jnp.int32)
    tabr = table.T.reshape(_NDT, 8000)
    out5 = _emb(xt, tabr)
    return jnp.transpose(out5, (2, 4, 0, 1, 3)).reshape(_BATCH, _T, _NDT * 8)
